# baseline (speedup 1.0000x reference)
import functools

import jax
import jax.numpy as jnp
from jax import lax
from jax.experimental import pallas as pl
from jax.experimental.pallas import tpu as pltpu
from jax.experimental.pallas import tpu_sc as plsc

_G = 512
_D = 32
_C = 4096
_NC = 2
_NS = 16
_GENES_PER_W = _G // (_NC * _NS)


def _sc_rows_body(sc_hbm, out_hbm, row_buf, row_sem):
    cid = lax.axis_index("c")
    sid = lax.axis_index("s")
    wid = sid * _NC + cid
    g0 = wid * _GENES_PER_W

    pltpu.sync_copy(sc_hbm.at[pl.ds(g0, _GENES_PER_W), :], row_buf)
    for k in range(_GENES_PER_W):
        pltpu.async_copy(
            row_buf.at[pl.ds(k, 1), :],
            out_hbm.at[g0 + k, pl.ds(_D, 1), :],
            row_sem,
        )
    for k in range(_GENES_PER_W):
        pltpu.make_async_copy(
            row_buf.at[pl.ds(k, 1), :],
            out_hbm.at[g0 + k, pl.ds(_D, 1), :],
            row_sem,
        ).wait()


def _tc_slabs_body(w_ref, buf_ref, out_ref, wt_ref):
    del buf_ref
    d = w_ref.shape[1]
    c = w_ref.shape[0]

    @pl.when(pl.program_id(0) == 0)
    def _():
        wt_ref[...] = jnp.transpose(w_ref[...], (1, 0))

    out_ref[...] = wt_ref[...][None, :, :]


def kernel(scRNA_count, embedding_weight):
    g, c = scRNA_count.shape
    c2, d = embedding_weight.shape
    assert (g, c, c2, d) == (_G, _C, _C, _D)

    mesh = plsc.VectorSubcoreMesh(core_axis_name="c", subcore_axis_name="s")
    rows_call = functools.partial(
        pl.kernel,
        mesh=mesh,
        out_type=jax.ShapeDtypeStruct((_G, _D + 1, _C), jnp.float32),
        scratch_types=[
            pltpu.VMEM((_GENES_PER_W, _C), jnp.float32),
            pltpu.SemaphoreType.DMA,
        ],
    )(_sc_rows_body)
    buf = rows_call(scRNA_count)

    return pl.pallas_call(
        _tc_slabs_body,
        grid=(g,),
        in_specs=[
            pl.BlockSpec((c, d), lambda i: (0, 0)),
            pl.BlockSpec(memory_space=pltpu.MemorySpace.HBM),
        ],
        out_specs=pl.BlockSpec((1, d, c), lambda i: (i, 0, 0)),
        out_shape=jax.ShapeDtypeStruct((g, d + 1, c), jnp.float32),
        scratch_shapes=[pltpu.VMEM((d, c), jnp.float32)],
        input_output_aliases={1: 0},
    )(embedding_weight, buf)


# speedup vs baseline: 1.1893x; 1.1893x over previous
import functools

import jax
import jax.numpy as jnp
from jax import lax
from jax.experimental import pallas as pl
from jax.experimental.pallas import tpu as pltpu
from jax.experimental.pallas import tpu_sc as plsc

_G = 512
_D = 32
_C = 4096
_NC = 2
_NS = 16
_L = 16
_NCOPY = 2
_CELLS_PER_SUB = _C // _NS
_GENES_PER_W = _G // (_NC * _NS)


def _sc_body(sc_hbm, w_hbm, out_hbm, wstage, wt_chunk, row_buf, wt_spmem,
             slab_sem, row_sem):
    cid = lax.axis_index("c")
    sid = lax.axis_index("s")
    wid = sid * _NC + cid

    cell0 = sid * _CELLS_PER_SUB
    pltpu.sync_copy(w_hbm.at[pl.ds(cell0, _CELLS_PER_SUB), :], wstage)
    lane = lax.iota(jnp.int32, _L)
    for d in range(_D):
        d_idx = jnp.full((_L,), d, jnp.int32)
        for cgrp in range(_CELLS_PER_SUB // _L):
            c_idx = lane + (cgrp * _L)
            v = plsc.load_gather(wstage, [c_idx, d_idx])
            wt_chunk[d, pl.ds(cgrp * _L, _L)] = v
    for r in range(_NCOPY):
        pltpu.sync_copy(
            wt_chunk,
            wt_spmem.at[pl.ds(r * _D, _D), pl.ds(cell0, _CELLS_PER_SUB)],
        )
    plsc.subcore_barrier()

    g0 = wid * _GENES_PER_W
    my_copy = lax.rem(sid, _NCOPY) * _D
    src = wt_spmem.at[pl.ds(my_copy, _D), :]
    pltpu.sync_copy(sc_hbm.at[pl.ds(g0, _GENES_PER_W), :], row_buf)
    for k in range(_GENES_PER_W):
        g = g0 + k
        pltpu.async_copy(src, out_hbm.at[g, pl.ds(0, _D), :], slab_sem)
        pltpu.async_copy(
            row_buf.at[pl.ds(k, 1), :], out_hbm.at[g, pl.ds(_D, 1), :], row_sem
        )
    for k in range(_GENES_PER_W):
        g = g0 + k
        pltpu.make_async_copy(
            src, out_hbm.at[g, pl.ds(0, _D), :], slab_sem
        ).wait()
        pltpu.make_async_copy(
            row_buf.at[pl.ds(k, 1), :], out_hbm.at[g, pl.ds(_D, 1), :], row_sem
        ).wait()


def kernel(scRNA_count, embedding_weight):
    g, c = scRNA_count.shape
    c2, d = embedding_weight.shape
    assert (g, c, c2, d) == (_G, _C, _C, _D)

    mesh = plsc.VectorSubcoreMesh(core_axis_name="c", subcore_axis_name="s")
    f = functools.partial(
        pl.kernel,
        mesh=mesh,
        out_type=jax.ShapeDtypeStruct((_G, _D + 1, _C), jnp.float32),
        compiler_params=pltpu.CompilerParams(needs_layout_passes=False),
        scratch_types=[
            pltpu.VMEM((_CELLS_PER_SUB, _D), jnp.float32),
            pltpu.VMEM((_D, _CELLS_PER_SUB), jnp.float32),
            pltpu.VMEM((_GENES_PER_W, _C), jnp.float32),
            pltpu.VMEM_SHARED((_NCOPY * _D, _C), jnp.float32),
            pltpu.SemaphoreType.DMA,
            pltpu.SemaphoreType.DMA,
        ],
    )(_sc_body)
    return f(scRNA_count, embedding_weight)


# speedup vs baseline: 1.3377x; 1.1248x over previous
import functools

import jax
import jax.numpy as jnp
from jax import lax
from jax.experimental import pallas as pl
from jax.experimental.pallas import tpu as pltpu
from jax.experimental.pallas import tpu_sc as plsc

_G = 512
_D = 32
_C = 4096
_NC = 2
_NS = 16
_L = 16
_CELLS_PER_SUB = _C // _NS
_NPAIR = _NC * (_NS // 2)
_GENES_PER_PAIR = _G // _NPAIR
_GENES_PER_W = _G // (_NC * _NS)
_ROWBATCH = 1


def _sc_body(sc_hbm, w_hbm, out_hbm, wstage, wt_chunk, row_buf, half,
             wt_spmem, slab_sem, row_sem):
    cid = lax.axis_index("c")
    sid = lax.axis_index("s")
    wid = sid * _NC + cid

    cell0 = sid * _CELLS_PER_SUB
    pltpu.sync_copy(w_hbm.at[pl.ds(cell0, _CELLS_PER_SUB), :], wstage)
    lane = lax.iota(jnp.int32, _L)
    for d in range(_D):
        d_idx = jnp.full((_L,), d, jnp.int32)
        for cgrp in range(_CELLS_PER_SUB // _L):
            c_idx = lane + (cgrp * _L)
            v = plsc.load_gather(wstage, [c_idx, d_idx])
            wt_chunk[d, pl.ds(cgrp * _L, _L)] = v
    pltpu.sync_copy(wt_chunk, wt_spmem.at[:, pl.ds(cell0, _CELLS_PER_SUB)])
    plsc.subcore_barrier()

    rhalf = lax.rem(sid, 2) * (_D // 2)
    pltpu.sync_copy(wt_spmem.at[pl.ds(rhalf, _D // 2), :], half)

    pair_id = cid * (_NS // 2) + lax.div(sid, 2)
    g0 = pair_id * _GENES_PER_PAIR
    for k in range(_GENES_PER_PAIR):
        pltpu.async_copy(
            half, out_hbm.at[g0 + k, pl.ds(rhalf, _D // 2), :], slab_sem
        )

    r0 = wid * _GENES_PER_W
    for b in range(_GENES_PER_W // _ROWBATCH):
        rb = r0 + b * _ROWBATCH
        pltpu.sync_copy(sc_hbm.at[pl.ds(rb, _ROWBATCH), :], row_buf)
        for k in range(_ROWBATCH):
            pltpu.async_copy(
                row_buf.at[pl.ds(k, 1), :],
                out_hbm.at[rb + k, pl.ds(_D, 1), :],
                row_sem,
            )
        for k in range(_ROWBATCH):
            pltpu.make_async_copy(
                row_buf.at[pl.ds(k, 1), :],
                out_hbm.at[rb + k, pl.ds(_D, 1), :],
                row_sem,
            ).wait()

    for k in range(_GENES_PER_PAIR):
        pltpu.make_async_copy(
            half, out_hbm.at[g0 + k, pl.ds(rhalf, _D // 2), :], slab_sem
        ).wait()


def kernel(scRNA_count, embedding_weight):
    g, c = scRNA_count.shape
    c2, d = embedding_weight.shape
    assert (g, c, c2, d) == (_G, _C, _C, _D)

    mesh = plsc.VectorSubcoreMesh(core_axis_name="c", subcore_axis_name="s")
    f = functools.partial(
        pl.kernel,
        mesh=mesh,
        out_type=jax.ShapeDtypeStruct((_G, _D + 1, _C), jnp.float32),
        compiler_params=pltpu.CompilerParams(needs_layout_passes=False),
        scratch_types=[
            pltpu.VMEM((_CELLS_PER_SUB, _D), jnp.float32),
            pltpu.VMEM((_D, _CELLS_PER_SUB), jnp.float32),
            pltpu.VMEM((_ROWBATCH, _C), jnp.float32),
            pltpu.VMEM((_D // 2, _C), jnp.float32),
            pltpu.VMEM_SHARED((_D, _C), jnp.float32),
            pltpu.SemaphoreType.DMA,
            pltpu.SemaphoreType.DMA,
        ],
    )(_sc_body)
    return f(scRNA_count, embedding_weight)


# speedup vs baseline: 1.4062x; 1.0513x over previous
import functools

import jax
import jax.numpy as jnp
from jax import lax
from jax.experimental import pallas as pl
from jax.experimental.pallas import tpu as pltpu
from jax.experimental.pallas import tpu_sc as plsc

_G = 512
_D = 32
_C = 4096
_NC = 2
_NS = 16
_GENES_PER_W = _G // (_NC * _NS)


def _sc_rows_body(sc_hbm, out_hbm, row_buf, row_sem):
    cid = lax.axis_index("c")
    sid = lax.axis_index("s")
    wid = sid * _NC + cid
    g0 = wid * _GENES_PER_W

    pltpu.sync_copy(sc_hbm.at[pl.ds(g0, _GENES_PER_W), :], row_buf)
    for k in range(_GENES_PER_W):
        pltpu.async_copy(
            row_buf.at[pl.ds(k, 1), :],
            out_hbm.at[g0 + k, pl.ds(_D, 1), :],
            row_sem,
        )
    for k in range(_GENES_PER_W):
        pltpu.make_async_copy(
            row_buf.at[pl.ds(k, 1), :],
            out_hbm.at[g0 + k, pl.ds(_D, 1), :],
            row_sem,
        ).wait()


def _tc_slabs_body(w_ref, buf_ref, out_ref, wt_ref):
    del buf_ref
    gblk = out_ref.shape[0]
    d = w_ref.shape[1]
    c = w_ref.shape[0]

    @pl.when(pl.program_id(0) == 0)
    def _():
        wt_ref[...] = jnp.transpose(w_ref[...], (1, 0))

    out_ref[...] = jnp.broadcast_to(wt_ref[...][None, :, :], (gblk, d, c))


def kernel(scRNA_count, embedding_weight):
    g, c = scRNA_count.shape
    c2, d = embedding_weight.shape
    assert (g, c, c2, d) == (_G, _C, _C, _D)

    mesh = plsc.VectorSubcoreMesh(core_axis_name="c", subcore_axis_name="s")
    rows_call = functools.partial(
        pl.kernel,
        mesh=mesh,
        out_type=jax.ShapeDtypeStruct((_G, _D + 1, _C), jnp.float32),
        scratch_types=[
            pltpu.VMEM((_GENES_PER_W, _C), jnp.float32),
            pltpu.SemaphoreType.DMA,
        ],
    )(_sc_rows_body)
    buf = rows_call(scRNA_count)

    gblk = 16
    return pl.pallas_call(
        _tc_slabs_body,
        grid=(g // gblk,),
        in_specs=[
            pl.BlockSpec((c, d), lambda i: (0, 0)),
            pl.BlockSpec(memory_space=pltpu.MemorySpace.HBM),
        ],
        out_specs=pl.BlockSpec((gblk, d, c), lambda i: (i, 0, 0)),
        out_shape=jax.ShapeDtypeStruct((g, d + 1, c), jnp.float32),
        scratch_shapes=[pltpu.VMEM((d, c), jnp.float32)],
        input_output_aliases={1: 0},
    )(embedding_weight, buf)
